# baseline (speedup 1.0000x reference)
import jax
import jax.numpy as jnp
from jax.experimental import pallas as pl
from jax.experimental.pallas import tpu as pltpu

B = 1024
N = 257
M = N - 1
C = 64
H = 64
L = 32
O = 64
BB = 16
NB = B // BB
CNT = float(B * H * N)


def _stats_row(sv, sqv):
    s = jnp.sum(sv)
    sq = jnp.sum(sqv)
    col = jax.lax.broadcasted_iota(jnp.int32, (1, 128), 1)
    return jnp.where(col == 0, s, 0.0) + jnp.where(col == 1, sq, 0.0)


def _read_stats(stat_ref, row):
    pv = stat_ref[row:row + 1, :]
    col = jax.lax.broadcasted_iota(jnp.int32, (1, 128), 1)
    s = jnp.sum(jnp.where(col == 0, pv, 0.0))
    sq = jnp.sum(jnp.where(col == 1, pv, 0.0))
    mu = s / CNT
    var = (sq - s * s / CNT) / (CNT - 1.0)
    inv = 1.0 / (jnp.sqrt(var) + 1e-5)
    return mu, inv


def _gather_conv(xb, ch, wT_ref, b_ref, iota_n, transposed):
    acc = None
    for k in range(3):
        ck = ch[:, k:k + 1]
        gk = (iota_n == ck).astype(jnp.float32)
        if transposed:
            ek = jax.lax.dot_general(
                gk, xb, (((1,), (1,)), ((), ())),
                preferred_element_type=jnp.float32)
        else:
            ek = jnp.dot(gk, xb, preferred_element_type=jnp.float32)
        term = jnp.dot(ek, wT_ref[k], preferred_element_type=jnp.float32)
        acc = term if acc is None else acc + term
    return acc + b_ref[...]


def _fused_kernel(nf_ref, ch_ref, z_ref, w1T_ref, b1_ref, w2T_ref, b2_ref,
                  wa_ref, wb_ref, mb1_ref, w2m_ref, mb2_ref,
                  out_ref, x_scr, stat_ref):
    p = pl.program_id(0)
    i = pl.program_id(1)
    iota_n = jax.lax.broadcasted_iota(jnp.int32, (M, N), 1)

    @pl.when(jnp.logical_and(p == 0, i == 0))
    def _():
        stat_ref[...] = jnp.zeros_like(stat_ref)

    @pl.when(p == 0)
    def _():
        sv = jnp.zeros((M, H), jnp.float32)
        sqv = jnp.zeros((M, H), jnp.float32)
        for b in range(BB):
            conv = _gather_conv(nf_ref[b], ch_ref[b], w1T_ref, b1_ref,
                                iota_n, True)
            s = (b % 2) * H
            x_scr[i, b // 2, 0:1, s:s + H] = jnp.zeros((1, H), jnp.bfloat16)
            x_scr[i, b // 2, 1:N, s:s + H] = conv.astype(jnp.bfloat16)
            sv = sv + conv
            sqv = sqv + conv * conv
        stat_ref[0:1, :] += _stats_row(sv, sqv)

    @pl.when(p == 1)
    def _():
        mu, inv = _read_stats(stat_ref, 0)
        sv = jnp.zeros((M, H), jnp.float32)
        sqv = jnp.zeros((M, H), jnp.float32)
        for b in range(BB):
            s = (b % 2) * H
            xb16 = x_scr[i, b // 2, :, s:s + H]
            xn = jnp.maximum((xb16.astype(jnp.float32) - mu) * inv, 0.0)
            conv = _gather_conv(xn, ch_ref[b], w2T_ref, b2_ref, iota_n, False)
            x_scr[i, b // 2, 0:1, s:s + H] = jnp.zeros((1, H), jnp.bfloat16)
            x_scr[i, b // 2, 1:N, s:s + H] = conv.astype(jnp.bfloat16)
            sv = sv + conv
            sqv = sqv + conv * conv
        stat_ref[1:2, :] += _stats_row(sv, sqv)

    @pl.when(p == 2)
    def _():
        mu, inv = _read_stats(stat_ref, 1)
        for b in range(BB):
            s = (b % 2) * H
            xb16 = x_scr[i, b // 2, :, s:s + H]
            xn = jnp.maximum((xb16.astype(jnp.float32) - mu) * inv, 0.0)
            zrow = z_ref[b:b + 1, :]
            t = jnp.dot(zrow, wb_ref[...], preferred_element_type=jnp.float32)
            h = jnp.dot(xn, wa_ref[...], preferred_element_type=jnp.float32)
            h = jnp.maximum(h + t + mb1_ref[...], 0.0)
            logits = jnp.dot(h, w2m_ref[...],
                             preferred_element_type=jnp.float32)
            out_ref[b] = logits + mb2_ref[...]


def _rep(shape):
    nd = len(shape)
    return pl.BlockSpec(shape, lambda p, i: (0,) * nd)


@jax.jit
def kernel(node_feats, children, z, conv1_w, conv1_b, conv2_w, conv2_b,
           mlp_w1, mlp_b1, mlp_w2, mlp_b2):
    ch = children[:, :, 0].reshape(B, M, 3)
    w1T = conv1_w.transpose(2, 1, 0)
    w2T = conv2_w.transpose(2, 1, 0)
    b1 = conv1_b.reshape(1, H)
    b2 = conv2_b.reshape(1, H)
    wa = mlp_w1[:H]
    wb = mlp_w1[H:]
    mb1 = mlp_b1.reshape(1, H)
    mb2 = mlp_b2.reshape(1, O)

    logits = pl.pallas_call(
        _fused_kernel,
        grid=(3, NB),
        in_specs=[
            pl.BlockSpec((BB, C, N),
                         lambda p, i: (jnp.where(p == 0, i, 0), 0, 0)),
            pl.BlockSpec((BB, M, 3),
                         lambda p, i: (jnp.where(p <= 1, i, 0), 0, 0)),
            pl.BlockSpec((BB, L), lambda p, i: (jnp.where(p == 2, i, 0), 0)),
            _rep((3, C, H)), _rep((1, H)),
            _rep((3, H, H)), _rep((1, H)),
            _rep((H, H)), _rep((L, H)), _rep((1, H)),
            _rep((H, O)), _rep((1, O)),
        ],
        out_specs=pl.BlockSpec((BB, N, O),
                               lambda p, i: (jnp.where(p == 2, i, 0), 0, 0)),
        out_shape=jax.ShapeDtypeStruct((B, N, O), jnp.float32),
        scratch_shapes=[
            pltpu.VMEM((NB, BB // 2, N, 2 * H), jnp.bfloat16),
            pltpu.VMEM((2, 128), jnp.float32),
        ],
    )(node_feats, ch, z, w1T, b1, w2T, b2, wa, wb, mb1, mlp_w2, mb2)

    return logits


# speedup vs baseline: 1.1099x; 1.1099x over previous
import jax
import jax.numpy as jnp
from jax.experimental import pallas as pl
from jax.experimental.pallas import tpu as pltpu

B = 1024
N = 257
M = N - 1
C = 64
H = 64
L = 32
O = 64
BB = 16
NB = B // BB
CNT = float(B * H * N)


def _stats_block(sv, sqv):
    s = jnp.sum(sv)
    sq = jnp.sum(sqv)
    col = jax.lax.broadcasted_iota(jnp.int32, (1, 128), 1)
    return jnp.where(col == 0, s, 0.0) + jnp.where(col == 1, sq, 0.0)


def _read_stats(part_ref):
    pv = part_ref[:, 0, :]
    col = jax.lax.broadcasted_iota(jnp.int32, (NB, 128), 1)
    s = jnp.sum(jnp.where(col == 0, pv, 0.0))
    sq = jnp.sum(jnp.where(col == 1, pv, 0.0))
    mu = s / CNT
    var = (sq - s * s / CNT) / (CNT - 1.0)
    inv = 1.0 / (jnp.sqrt(var) + 1e-5)
    return mu, inv


def _conv1_kernel(x_ref, ch_ref, wT_ref, b_ref, out_ref, part_ref):
    sv = jnp.zeros((M, H), jnp.float32)
    sqv = jnp.zeros((M, H), jnp.float32)
    iota_n = jax.lax.broadcasted_iota(jnp.int32, (M, N), 1)
    for b in range(BB):
        xcm = x_ref[b]
        ch = ch_ref[b]
        acc = None
        for k in range(3):
            ck = ch[:, k:k + 1]
            gk = (iota_n == ck).astype(jnp.float32)
            ek = jax.lax.dot_general(
                gk, xcm, (((1,), (1,)), ((), ())),
                preferred_element_type=jnp.float32)
            term = jnp.dot(ek, wT_ref[k], preferred_element_type=jnp.float32)
            acc = term if acc is None else acc + term
        conv = acc + b_ref[...]
        out_ref[b, 0:1, :] = jnp.zeros((1, H), jnp.bfloat16)
        out_ref[b, 1:N, :] = conv.astype(jnp.bfloat16)
        sv = sv + conv
        sqv = sqv + conv * conv
    part_ref[0] = _stats_block(sv, sqv)


def _conv2_kernel(x_ref, ch_ref, part_in_ref, wT_ref, b_ref, out_ref,
                  part_ref):
    mu, inv = _read_stats(part_in_ref)
    sv = jnp.zeros((M, H), jnp.float32)
    sqv = jnp.zeros((M, H), jnp.float32)
    iota_n = jax.lax.broadcasted_iota(jnp.int32, (M, N), 1)
    for b in range(BB):
        xn = jnp.maximum((x_ref[b].astype(jnp.float32) - mu) * inv, 0.0)
        ch = ch_ref[b]
        acc = None
        for k in range(3):
            ck = ch[:, k:k + 1]
            gk = (iota_n == ck).astype(jnp.float32)
            ek = jnp.dot(gk, xn, preferred_element_type=jnp.float32)
            term = jnp.dot(ek, wT_ref[k], preferred_element_type=jnp.float32)
            acc = term if acc is None else acc + term
        conv = acc + b_ref[...]
        out_ref[b, 0:1, :] = jnp.zeros((1, H), jnp.bfloat16)
        out_ref[b, 1:N, :] = conv.astype(jnp.bfloat16)
        sv = sv + conv
        sqv = sqv + conv * conv
    part_ref[0] = _stats_block(sv, sqv)


def _mlp_kernel(x_ref, part_in_ref, z_ref, wa_ref, wb_ref, b1_ref, w2_ref,
                b2_ref, out_ref):
    mu, inv = _read_stats(part_in_ref)
    for b in range(BB):
        xn = jnp.maximum((x_ref[b].astype(jnp.float32) - mu) * inv, 0.0)
        zrow = z_ref[b:b + 1, :]
        t = jnp.dot(zrow, wb_ref[...], preferred_element_type=jnp.float32)
        h = jnp.dot(xn, wa_ref[...], preferred_element_type=jnp.float32)
        h = jnp.maximum(h + t + b1_ref[...], 0.0)
        logits = jnp.dot(h, w2_ref[...], preferred_element_type=jnp.float32)
        out_ref[b] = logits + b2_ref[...]


def _rep(shape):
    nd = len(shape)
    return pl.BlockSpec(shape, lambda i: (0,) * nd)


_PARALLEL = pltpu.CompilerParams(dimension_semantics=("parallel",))


@jax.jit
def kernel(node_feats, children, z, conv1_w, conv1_b, conv2_w, conv2_b,
           mlp_w1, mlp_b1, mlp_w2, mlp_b2):
    grid = (NB,)
    ch = children[:, :, 0].reshape(B, M, 3)
    w1T = conv1_w.transpose(2, 1, 0)
    w2T = conv2_w.transpose(2, 1, 0)
    b1 = conv1_b.reshape(1, H)
    b2 = conv2_b.reshape(1, H)
    wa = mlp_w1[:H]
    wb = mlp_w1[H:]
    mb1 = mlp_b1.reshape(1, H)
    mb2 = mlp_b2.reshape(1, O)

    x_spec = pl.BlockSpec((BB, N, C), lambda i: (i, 0, 0))
    ch_spec = pl.BlockSpec((BB, M, 3), lambda i: (i, 0, 0))
    pout_spec = pl.BlockSpec((1, 1, 128), lambda i: (i, 0, 0))
    part_shape = jax.ShapeDtypeStruct((NB, 1, 128), jnp.float32)

    x1, part1 = pl.pallas_call(
        _conv1_kernel,
        grid=grid,
        in_specs=[pl.BlockSpec((BB, C, N), lambda i: (i, 0, 0)),
                  ch_spec, _rep((3, C, H)), _rep((1, H))],
        out_specs=[x_spec, pout_spec],
        out_shape=[jax.ShapeDtypeStruct((B, N, H), jnp.bfloat16), part_shape],
        compiler_params=_PARALLEL,
    )(node_feats, ch, w1T, b1)

    x2, part2 = pl.pallas_call(
        _conv2_kernel,
        grid=grid,
        in_specs=[x_spec, ch_spec, _rep((NB, 1, 128)), _rep((3, H, H)),
                  _rep((1, H))],
        out_specs=[x_spec, pout_spec],
        out_shape=[jax.ShapeDtypeStruct((B, N, H), jnp.bfloat16), part_shape],
        compiler_params=_PARALLEL,
    )(x1, ch, part1, w2T, b2)

    logits = pl.pallas_call(
        _mlp_kernel,
        grid=grid,
        in_specs=[
            x_spec, _rep((NB, 1, 128)),
            pl.BlockSpec((BB, L), lambda i: (i, 0)),
            _rep((H, H)), _rep((L, H)), _rep((1, H)),
            _rep((H, O)), _rep((1, O)),
        ],
        out_specs=pl.BlockSpec((BB, N, O), lambda i: (i, 0, 0)),
        out_shape=jax.ShapeDtypeStruct((B, N, O), jnp.float32),
        compiler_params=_PARALLEL,
    )(x2, part2, z, wa, wb, mb1, mlp_w2, mb2)

    return logits
